# baseline (speedup 1.0000x reference)
import functools

import jax
import jax.numpy as jnp
from jax.experimental import pallas as pl
from jax.experimental.pallas import tpu as pltpu

TILE_N = 512


def _moe_body(x_ref, wg_ref, bg_ref, w1_ref, b1_ref, w2_ref, b2_ref,
              gamma_ref, beta_ref, out_ref, lb_ref, acc_ref, gsum_ref,
              *, nt, n_experts, n_tokens):
    t = pl.program_id(0)
    e = pl.program_id(1)
    x = x_ref[...]

    logits = jnp.dot(x, wg_ref[...], preferred_element_type=jnp.float32)
    logits = logits + bg_ref[...]
    iota_e = jax.lax.broadcasted_iota(jnp.int32, logits.shape, 1)
    m1 = jnp.max(logits, axis=1, keepdims=True)
    idx1 = jnp.min(jnp.where(logits == m1, iota_e, n_experts), axis=1,
                   keepdims=True)
    l2 = jnp.where(iota_e == idx1, -jnp.inf, logits)
    m2 = jnp.max(l2, axis=1, keepdims=True)
    idx2 = jnp.min(jnp.where(l2 == m2, iota_e, n_experts), axis=1,
                   keepdims=True)
    e2 = jnp.exp(m2 - m1)
    g1 = 1.0 / (1.0 + e2)
    g2 = e2 * g1

    @pl.when(e == 0)
    def _():
        sg = g1 * (iota_e == idx1) + g2 * (iota_e == idx2)
        sg_sum = jnp.sum(sg, axis=0, keepdims=True)
        prev = jnp.where(t == 0, jnp.zeros_like(sg_sum), gsum_ref[...])
        gsum_ref[...] = prev + sg_sum

    ge = g1 * (idx1 == e) + g2 * (idx2 == e)

    h = jnp.dot(x, w1_ref[0], preferred_element_type=jnp.float32)
    h = jnp.maximum(h + b1_ref[0], 0.0)
    eo = jnp.dot(h, w2_ref[0], preferred_element_type=jnp.float32)
    eo = eo + b2_ref[0]
    contrib = ge * eo
    acc = jnp.where(e == 0, contrib, acc_ref[...] + contrib)
    acc_ref[...] = acc

    @pl.when(e == n_experts - 1)
    def _():
        y = acc + x
        mu = jnp.mean(y, axis=1, keepdims=True)
        yc = y - mu
        var = jnp.mean(yc * yc, axis=1, keepdims=True)
        out_ref[...] = yc * jax.lax.rsqrt(var + 1e-5) * gamma_ref[...] \
            + beta_ref[...]

    @pl.when((t == nt - 1) & (e == n_experts - 1))
    def _():
        d_i = gsum_ref[...] / n_tokens
        lb_ref[...] = jnp.sum(d_i * jnp.log(d_i + 1e-8), keepdims=True
                              ).reshape(1, 1)


def kernel(x, W_gate, b_gate, W1, b1, W2, b2, gamma, beta):
    n, d = x.shape
    e_num = W_gate.shape[1]
    h_dim = W1.shape[2]
    nt = n // TILE_N

    body = functools.partial(_moe_body, nt=nt, n_experts=e_num, n_tokens=n)
    out, lb = pl.pallas_call(
        body,
        grid=(nt, e_num),
        in_specs=[
            pl.BlockSpec((TILE_N, d), lambda t, e: (t, 0)),
            pl.BlockSpec((d, e_num), lambda t, e: (0, 0)),
            pl.BlockSpec((1, e_num), lambda t, e: (0, 0)),
            pl.BlockSpec((1, d, h_dim), lambda t, e: (e, 0, 0)),
            pl.BlockSpec((1, 1, h_dim), lambda t, e: (e, 0, 0)),
            pl.BlockSpec((1, h_dim, d), lambda t, e: (e, 0, 0)),
            pl.BlockSpec((1, 1, d), lambda t, e: (e, 0, 0)),
            pl.BlockSpec((1, d), lambda t, e: (0, 0)),
            pl.BlockSpec((1, d), lambda t, e: (0, 0)),
        ],
        out_specs=[
            pl.BlockSpec((TILE_N, d), lambda t, e: (t, 0)),
            pl.BlockSpec((1, 1), lambda t, e: (0, 0)),
        ],
        out_shape=[
            jax.ShapeDtypeStruct((n, d), jnp.float32),
            jax.ShapeDtypeStruct((1, 1), jnp.float32),
        ],
        scratch_shapes=[
            pltpu.VMEM((TILE_N, d), jnp.float32),
            pltpu.VMEM((1, e_num), jnp.float32),
        ],
    )(x, W_gate, b_gate.reshape(1, e_num), W1,
      b1.reshape(e_num, 1, h_dim), W2, b2.reshape(e_num, 1, d),
      gamma.reshape(1, d), beta.reshape(1, d))
    return out, lb[0, 0]
